# Initial kernel scaffold; baseline (speedup 1.0000x reference)
import jax
import jax.numpy as jnp
from jax import lax
from jax.experimental import pallas as pl
from jax.experimental.pallas import tpu as pltpu
from jax.experimental.pallas import tpu_sc as plsc

PAD = 1
BATCH = 4096
SEQ = 200
DIM = 64
NCORES = 2
NSUB = 16
NW = NCORES * NSUB
BPW = BATCH // NW
LP = 208
CH = 104
NCH = (2 * LP) // CH
UNROLL = 8
LANE = 16


def _load_idx(src_hbm, trg_hbm, idxbuf, row):
    pltpu.sync_copy(src_hbm.at[row], idxbuf.at[pl.ds(0, SEQ)])
    pltpu.sync_copy(trg_hbm.at[row], idxbuf.at[pl.ds(LP, SEQ)])


def _fire(emb_hbm, idxbuf, rowsbuf, sem):
    for j in range(NCH):
        pltpu.async_copy(
            emb_hbm.at[idxbuf.at[pl.ds(j * CH, CH)]],
            rowsbuf.at[pl.ds(j * CH, CH)],
            sem,
        )


def _drain(emb_hbm, idxbuf, rowsbuf, sem):
    for j in range(NCH):
        pltpu.make_async_copy(
            emb_hbm.at[idxbuf.at[pl.ds(j * CH, CH)]],
            rowsbuf.at[pl.ds(j * CH, CH)],
            sem,
        ).wait()


def _sum_side(rowsbuf, lo):
    zero = jnp.zeros((LANE,), jnp.float32)

    def body(i, acc):
        a0, a1, a2, a3 = acc
        for u in range(UNROLL):
            r = lo + i * UNROLL + u
            a0 = a0 + rowsbuf[r, pl.ds(0, LANE)]
            a1 = a1 + rowsbuf[r, pl.ds(16, LANE)]
            a2 = a2 + rowsbuf[r, pl.ds(32, LANE)]
            a3 = a3 + rowsbuf[r, pl.ds(48, LANE)]
        return (a0, a1, a2, a3)

    return lax.fori_loop(0, LP // UNROLL, body, (zero, zero, zero, zero))


def _count_side(idxbuf, lo):
    one = jnp.ones((LANE,), jnp.float32)
    zero = jnp.zeros((LANE,), jnp.float32)
    c = zero
    for i in range(LP // LANE):
        c = c + jnp.where(idxbuf[pl.ds(lo + i * LANE, LANE)] != PAD, one, zero)
    return jnp.sum(c)


def _consume(idxbuf, rowsbuf, wregs, bscal, outv, r):
    xs = _sum_side(rowsbuf, 0)
    ys = _sum_side(rowsbuf, LP)
    ns = _count_side(idxbuf, 0)
    nt = _count_side(idxbuf, LP)
    t = jnp.zeros((LANE,), jnp.float32)
    for c in range(4):
        t = t + (xs[c] / ns) * wregs[c] + (ys[c] / nt) * wregs[4 + c]
    outv[r] = jnp.sum(t) + bscal


def _sc_body(src_hbm, trg_hbm, emb_hbm, wb_hbm, out_hbm,
             idx0, idx1, rows0, rows1, wv, outv, sem0, sem1):
    cid = lax.axis_index("c")
    sid = lax.axis_index("s")
    wid = sid * NCORES + cid
    base = wid * BPW

    pltpu.sync_copy(wb_hbm, wv)
    wregs = [wv[pl.ds(c * LANE, LANE)] for c in range(8)]
    bscal = wv[2 * DIM]

    pad16 = jnp.full((LANE,), PAD, jnp.int32)
    for buf in (idx0, idx1):
        buf[pl.ds(LP - LANE, LANE)] = pad16
        buf[pl.ds(2 * LP - LANE, LANE)] = pad16

    _load_idx(src_hbm, trg_hbm, idx0, base)
    _fire(emb_hbm, idx0, rows0, sem0)

    def pair_body(k, carry):
        r0 = base + 2 * k
        _load_idx(src_hbm, trg_hbm, idx1, r0 + 1)
        _fire(emb_hbm, idx1, rows1, sem1)
        _drain(emb_hbm, idx0, rows0, sem0)
        _consume(idx0, rows0, wregs, bscal, outv, 2 * k)

        @pl.when(k < BPW // 2 - 1)
        def _():
            _load_idx(src_hbm, trg_hbm, idx0, r0 + 2)
            _fire(emb_hbm, idx0, rows0, sem0)

        _drain(emb_hbm, idx1, rows1, sem1)
        _consume(idx1, rows1, wregs, bscal, outv, 2 * k + 1)
        return carry

    lax.fori_loop(0, BPW // 2, pair_body, 0)

    for i in range(BPW // LANE):
        v = outv[pl.ds(i * LANE, LANE)]
        outv[pl.ds(i * LANE, LANE)] = 1.0 / (1.0 + jnp.exp(-v))

    pltpu.sync_copy(outv, out_hbm.at[pl.ds(base, BPW)])


@jax.jit
def kernel(src_tokens, trg_tokens, emb, W, b):
    wb = jnp.concatenate(
        [W.reshape(-1), b.reshape(-1),
         jnp.zeros((7,), jnp.float32)]).astype(jnp.float32)
    mesh = plsc.VectorSubcoreMesh(
        core_axis_name="c", subcore_axis_name="s",
        num_cores=NCORES, num_subcores=NSUB)
    out = pl.kernel(
        _sc_body,
        out_type=jax.ShapeDtypeStruct((BATCH,), jnp.float32),
        mesh=mesh,
        scratch_types=[
            pltpu.VMEM((2 * LP,), jnp.int32),
            pltpu.VMEM((2 * LP,), jnp.int32),
            pltpu.VMEM((2 * LP, DIM), jnp.float32),
            pltpu.VMEM((2 * LP, DIM), jnp.float32),
            pltpu.VMEM((2 * DIM + 8,), jnp.float32),
            pltpu.VMEM((BPW,), jnp.float32),
            pltpu.SemaphoreType.DMA,
            pltpu.SemaphoreType.DMA,
        ],
    )(src_tokens.astype(jnp.int32), trg_tokens.astype(jnp.int32), emb, wb)
    return out.reshape(BATCH, 1)


# speedup vs baseline: 3.7392x; 3.7392x over previous
import jax
import jax.numpy as jnp
from jax import lax
from jax.experimental import pallas as pl
from jax.experimental.pallas import tpu as pltpu
from jax.experimental.pallas import tpu_sc as plsc

PAD = 1
BATCH = 4096
SEQ = 200
DIM = 64
NCORES = 2
NSUB = 16
NW = NCORES * NSUB
BPW = BATCH // NW
C0 = 96
C1 = 112
LP = C0 + C1
UNROLL = 8
LANE = 16


def _load_idx(src_hbm, trg_hbm, chunks, row):
    s0, s1, t0, t1 = chunks
    off = pl.multiple_of(row * SEQ, 8)
    pltpu.sync_copy(src_hbm.at[pl.ds(off, C0)], s0)
    pltpu.sync_copy(src_hbm.at[pl.ds(off + C0, SEQ - C0)], s1.at[pl.ds(0, SEQ - C0)])
    pltpu.sync_copy(trg_hbm.at[pl.ds(off, C0)], t0)
    pltpu.sync_copy(trg_hbm.at[pl.ds(off + C0, SEQ - C0)], t1.at[pl.ds(0, SEQ - C0)])


def _gather_args(emb_hbm, chunks, rowsbuf):
    s0, s1, t0, t1 = chunks
    return (
        (emb_hbm.at[s0], rowsbuf.at[pl.ds(0, C0)]),
        (emb_hbm.at[s1], rowsbuf.at[pl.ds(C0, C1)]),
        (emb_hbm.at[t0], rowsbuf.at[pl.ds(LP, C0)]),
        (emb_hbm.at[t1], rowsbuf.at[pl.ds(LP + C0, C1)]),
    )


def _fire(emb_hbm, chunks, rowsbuf, sem):
    for src, dst in _gather_args(emb_hbm, chunks, rowsbuf):
        pltpu.async_copy(src, dst, sem)


def _drain(emb_hbm, chunks, rowsbuf, sem):
    for src, dst in _gather_args(emb_hbm, chunks, rowsbuf):
        pltpu.make_async_copy(src, dst, sem).wait()


def _sum_side(rowsbuf, lo):
    zero = jnp.zeros((LANE,), jnp.float32)

    def body(i, acc):
        a0, a1, a2, a3 = acc
        for u in range(UNROLL):
            r = lo + i * UNROLL + u
            a0 = a0 + rowsbuf[r, pl.ds(0, LANE)]
            a1 = a1 + rowsbuf[r, pl.ds(16, LANE)]
            a2 = a2 + rowsbuf[r, pl.ds(32, LANE)]
            a3 = a3 + rowsbuf[r, pl.ds(48, LANE)]
        return (a0, a1, a2, a3)

    return lax.fori_loop(0, LP // UNROLL, body, (zero, zero, zero, zero))


def _count_side(c0, c1):
    one = jnp.ones((LANE,), jnp.float32)
    zero = jnp.zeros((LANE,), jnp.float32)
    c = zero
    for i in range(C0 // LANE):
        c = c + jnp.where(c0[pl.ds(i * LANE, LANE)] != PAD, one, zero)
    for i in range(C1 // LANE):
        c = c + jnp.where(c1[pl.ds(i * LANE, LANE)] != PAD, one, zero)
    return c


def _consume(chunks, rowsbuf, wregs, stage, rr):
    s0, s1, t0, t1 = chunks
    xs = _sum_side(rowsbuf, 0)
    ys = _sum_side(rowsbuf, LP)
    u = jnp.zeros((LANE,), jnp.float32)
    v = jnp.zeros((LANE,), jnp.float32)
    for c in range(4):
        u = u + xs[c] * wregs[c]
        v = v + ys[c] * wregs[4 + c]
    stage[rr, pl.ds(0, LANE)] = u
    stage[rr, pl.ds(LANE, LANE)] = v
    stage[rr, pl.ds(2 * LANE, LANE)] = _count_side(s0, s1)
    stage[rr, pl.ds(3 * LANE, LANE)] = _count_side(t0, t1)


def _reduce_group(stage, bsplat, outv, goff):
    lane = lax.iota(jnp.int32, LANE)
    acc = [jnp.zeros((LANE,), jnp.float32) for _ in range(4)]
    for c in range(LANE):
        for q in range(4):
            acc[q] = acc[q] + plsc.load_gather(
                stage, [lane, jnp.full((LANE,), q * LANE + c, jnp.int32)])
    z = acc[0] / acc[2] + acc[1] / acc[3] + bsplat
    plsc.store_scatter(outv, [goff + lane], z)


def _sc_body(src_hbm, trg_hbm, emb_hbm, wb_hbm, out_hbm,
             s0a, s1a, t0a, t1a, s0b, s1b, t0b, t1b,
             rows0, rows1, wv, stage, outv, sem0, sem1):
    chunks0 = (s0a, s1a, t0a, t1a)
    chunks1 = (s0b, s1b, t0b, t1b)
    cid = lax.axis_index("c")
    sid = lax.axis_index("s")
    wid = sid * NCORES + cid
    base = wid * BPW

    pltpu.sync_copy(wb_hbm, wv)
    wregs = [wv[pl.ds(c * LANE, LANE)] for c in range(8)]
    bsplat = plsc.load_gather(wv, [jnp.full((LANE,), 2 * DIM, jnp.int32)])

    pad16 = jnp.full((LANE,), PAD, jnp.int32)
    for buf in (s1a, t1a, s1b, t1b):
        buf[pl.ds(C1 - LANE, LANE)] = pad16

    _load_idx(src_hbm, trg_hbm, chunks0, base)
    _fire(emb_hbm, chunks0, rows0, sem0)

    def pair_body(k, carry):
        r0 = base + 2 * k
        rr = (2 * k) & (LANE - 1)
        _load_idx(src_hbm, trg_hbm, chunks1, r0 + 1)
        _fire(emb_hbm, chunks1, rows1, sem1)
        _drain(emb_hbm, chunks0, rows0, sem0)
        _consume(chunks0, rows0, wregs, stage, rr)

        @pl.when(k < BPW // 2 - 1)
        def _():
            _load_idx(src_hbm, trg_hbm, chunks0, r0 + 2)
            _fire(emb_hbm, chunks0, rows0, sem0)

        _drain(emb_hbm, chunks1, rows1, sem1)
        _consume(chunks1, rows1, wregs, stage, rr + 1)

        @pl.when((k & 7) == 7)
        def _():
            _reduce_group(stage, bsplat, outv, (k // 8) * LANE)

        return carry

    lax.fori_loop(0, BPW // 2, pair_body, 0)

    for i in range(BPW // LANE):
        v = outv[pl.ds(i * LANE, LANE)]
        outv[pl.ds(i * LANE, LANE)] = 1.0 / (1.0 + jnp.exp(-v))

    pltpu.sync_copy(outv, out_hbm.at[pl.ds(base, BPW)])


@jax.jit
def kernel(src_tokens, trg_tokens, emb, W, b):
    wb = jnp.concatenate(
        [W.reshape(-1), b.reshape(-1),
         jnp.zeros((15,), jnp.float32)]).astype(jnp.float32)
    mesh = plsc.VectorSubcoreMesh(
        core_axis_name="c", subcore_axis_name="s",
        num_cores=NCORES, num_subcores=NSUB)
    idx_scratch = [
        pltpu.VMEM((C0,), jnp.int32), pltpu.VMEM((C1,), jnp.int32),
        pltpu.VMEM((C0,), jnp.int32), pltpu.VMEM((C1,), jnp.int32),
    ]
    out = pl.kernel(
        _sc_body,
        out_type=jax.ShapeDtypeStruct((BATCH,), jnp.float32),
        mesh=mesh,
        compiler_params=pltpu.CompilerParams(
            needs_layout_passes=False, use_tc_tiling_on_sc=False),
        scratch_types=idx_scratch * 2 + [
            pltpu.VMEM((2 * LP, DIM), jnp.float32),
            pltpu.VMEM((2 * LP, DIM), jnp.float32),
            pltpu.VMEM((2 * DIM + LANE,), jnp.float32),
            pltpu.VMEM((LANE, 4 * LANE), jnp.float32),
            pltpu.VMEM((BPW,), jnp.float32),
            pltpu.SemaphoreType.DMA,
            pltpu.SemaphoreType.DMA,
        ],
    )(src_tokens.astype(jnp.int32).reshape(-1),
      trg_tokens.astype(jnp.int32).reshape(-1), emb, wb)
    return out.reshape(BATCH, 1)


# speedup vs baseline: 19.2616x; 5.1512x over previous
import jax
import jax.numpy as jnp
from jax import lax
from jax.experimental import pallas as pl
from jax.experimental.pallas import tpu as pltpu
from jax.experimental.pallas import tpu_sc as plsc

PAD = 1
BATCH = 4096
SEQ = 200
DIM = 64
NCORES = 2
NSUB = 16
NW = NCORES * NSUB
BPW = BATCH // NW
NBLK = BPW // 2
BI = 4 * SEQ
CH = 128
UNROLL = 8
LANE = 16


def _load_block(src_hbm, trg_hbm, idxbuf, blk_row):
    off = pl.multiple_of(blk_row * SEQ, 8)
    pltpu.sync_copy(src_hbm.at[pl.ds(off, 2 * SEQ)], idxbuf.at[pl.ds(0, 2 * SEQ)])
    pltpu.sync_copy(trg_hbm.at[pl.ds(off, 2 * SEQ)], idxbuf.at[pl.ds(2 * SEQ, 2 * SEQ)])


def _gather_args(emb_hbm, idxbuf, rowsbuf):
    args = []
    for j in range(BI // CH):
        args.append((emb_hbm.at[idxbuf.at[pl.ds(j * CH, CH)]],
                     rowsbuf.at[pl.ds(j * CH, CH)]))
    rem = BI % CH
    if rem:
        args.append((emb_hbm.at[idxbuf.at[pl.ds(BI - rem, rem)]],
                     rowsbuf.at[pl.ds(BI - rem, rem)]))
    return args


def _fire(emb_hbm, idxbuf, rowsbuf, sem):
    for src, dst in _gather_args(emb_hbm, idxbuf, rowsbuf):
        pltpu.async_copy(src, dst, sem)


def _drain(emb_hbm, idxbuf, rowsbuf, sem):
    for src, dst in _gather_args(emb_hbm, idxbuf, rowsbuf):
        pltpu.make_async_copy(src, dst, sem).wait()


def _sum_span(rowsbuf, lo):
    zero = jnp.zeros((LANE,), jnp.float32)

    def body(i, acc):
        a0, a1, a2, a3 = acc
        for u in range(UNROLL):
            r = lo + i * UNROLL + u
            a0 = a0 + rowsbuf[r, pl.ds(0, LANE)]
            a1 = a1 + rowsbuf[r, pl.ds(16, LANE)]
            a2 = a2 + rowsbuf[r, pl.ds(32, LANE)]
            a3 = a3 + rowsbuf[r, pl.ds(48, LANE)]
        return (a0, a1, a2, a3)

    return lax.fori_loop(0, SEQ // UNROLL, body, (zero, zero, zero, zero))


def _count_span(idxbuf, lo):
    one = jnp.ones((LANE,), jnp.float32)
    zero = jnp.zeros((LANE,), jnp.float32)
    c = zero
    for i in range(SEQ // LANE):
        c = c + jnp.where(idxbuf[pl.ds(lo + i * LANE, LANE)] != PAD, one, zero)
    tail = idxbuf[pl.ds(lo + SEQ - LANE, LANE)]
    lane = lax.iota(jnp.int32, LANE)
    c = c + jnp.where((tail != PAD) & (lane >= LANE - SEQ % LANE), one, zero)
    return c


def _consume_row(idxbuf, rowsbuf, wregs, stage, half, rr):
    xs = _sum_span(rowsbuf, half * SEQ)
    ys = _sum_span(rowsbuf, 2 * SEQ + half * SEQ)
    u = jnp.zeros((LANE,), jnp.float32)
    v = jnp.zeros((LANE,), jnp.float32)
    for c in range(4):
        u = u + xs[c] * wregs[c]
        v = v + ys[c] * wregs[4 + c]
    stage[rr, pl.ds(0, LANE)] = u
    stage[rr, pl.ds(LANE, LANE)] = v
    stage[rr, pl.ds(2 * LANE, LANE)] = _count_span(idxbuf, half * SEQ)
    stage[rr, pl.ds(3 * LANE, LANE)] = _count_span(idxbuf, 2 * SEQ + half * SEQ)


def _reduce_group(stage, bsplat, outv, goff):
    lane = lax.iota(jnp.int32, LANE)
    acc = [jnp.zeros((LANE,), jnp.float32) for _ in range(4)]
    for c in range(LANE):
        for q in range(4):
            acc[q] = acc[q] + plsc.load_gather(
                stage, [lane, jnp.full((LANE,), q * LANE + c, jnp.int32)])
    z = acc[0] / acc[2] + acc[1] / acc[3] + bsplat
    plsc.store_scatter(outv, [goff + lane], z)


def _sc_body(src_hbm, trg_hbm, emb_hbm, wb_hbm, out_hbm,
             idx0, idx1, rows0, rows1, wv, stage, outv, sem0, sem1):
    cid = lax.axis_index("c")
    sid = lax.axis_index("s")
    wid = sid * NCORES + cid
    base = wid * BPW

    pltpu.sync_copy(wb_hbm, wv)
    wregs = [wv[pl.ds(c * LANE, LANE)] for c in range(8)]
    bsplat = plsc.load_gather(wv, [jnp.full((LANE,), 2 * DIM, jnp.int32)])

    _load_block(src_hbm, trg_hbm, idx0, base)
    _fire(emb_hbm, idx0, rows0, sem0)

    def ring_body(j, carry):
        k0 = 2 * j
        rr0 = (2 * k0) & (LANE - 1)
        _load_block(src_hbm, trg_hbm, idx1, base + 2 * k0 + 2)
        _fire(emb_hbm, idx1, rows1, sem1)
        _drain(emb_hbm, idx0, rows0, sem0)
        _consume_row(idx0, rows0, wregs, stage, 0, rr0)
        _consume_row(idx0, rows0, wregs, stage, 1, rr0 + 1)

        @pl.when(j < NBLK // 2 - 1)
        def _():
            _load_block(src_hbm, trg_hbm, idx0, base + 2 * k0 + 4)
            _fire(emb_hbm, idx0, rows0, sem0)

        rr1 = (2 * k0 + 2) & (LANE - 1)
        _drain(emb_hbm, idx1, rows1, sem1)
        _consume_row(idx1, rows1, wregs, stage, 0, rr1)
        _consume_row(idx1, rows1, wregs, stage, 1, rr1 + 1)

        @pl.when((k0 & 7) == 6)
        def _():
            _reduce_group(stage, bsplat, outv, (k0 // 8) * LANE)

        return carry

    lax.fori_loop(0, NBLK // 2, ring_body, 0)

    for i in range(BPW // LANE):
        v = outv[pl.ds(i * LANE, LANE)]
        outv[pl.ds(i * LANE, LANE)] = 1.0 / (1.0 + jnp.exp(-v))

    pltpu.sync_copy(outv, out_hbm.at[pl.ds(base, BPW)])


@jax.jit
def kernel(src_tokens, trg_tokens, emb, W, b):
    wb = jnp.concatenate(
        [W.reshape(-1), b.reshape(-1),
         jnp.zeros((15,), jnp.float32)]).astype(jnp.float32)
    mesh = plsc.VectorSubcoreMesh(
        core_axis_name="c", subcore_axis_name="s",
        num_cores=NCORES, num_subcores=NSUB)
    out = pl.kernel(
        _sc_body,
        out_type=jax.ShapeDtypeStruct((BATCH,), jnp.float32),
        mesh=mesh,
        compiler_params=pltpu.CompilerParams(
            needs_layout_passes=False, use_tc_tiling_on_sc=False),
        scratch_types=[
            pltpu.VMEM((BI,), jnp.int32),
            pltpu.VMEM((BI,), jnp.int32),
            pltpu.VMEM((BI, DIM), jnp.float32),
            pltpu.VMEM((BI, DIM), jnp.float32),
            pltpu.VMEM((2 * DIM + LANE,), jnp.float32),
            pltpu.VMEM((LANE, 4 * LANE), jnp.float32),
            pltpu.VMEM((BPW,), jnp.float32),
            pltpu.SemaphoreType.DMA,
            pltpu.SemaphoreType.DMA,
        ],
    )(src_tokens.astype(jnp.int32).reshape(-1),
      trg_tokens.astype(jnp.int32).reshape(-1), emb, wb)
    return out.reshape(BATCH, 1)


# speedup vs baseline: 22.8095x; 1.1842x over previous
import jax
import jax.numpy as jnp
from jax import lax
from jax.experimental import pallas as pl
from jax.experimental.pallas import tpu as pltpu
from jax.experimental.pallas import tpu_sc as plsc

PAD = 1
BATCH = 4096
SEQ = 200
DIM = 64
NCORES = 2
NSUB = 16
NW = NCORES * NSUB
BPW = BATCH // NW
NBLK = BPW // 2
BI = 4 * SEQ
CH = 128
UNROLL = 8
LANE = 16


def _load_block(src_hbm, trg_hbm, idxbuf, blk_row):
    off = pl.multiple_of(blk_row * SEQ, 8)
    pltpu.sync_copy(src_hbm.at[pl.ds(off, 2 * SEQ)], idxbuf.at[pl.ds(0, 2 * SEQ)])
    pltpu.sync_copy(trg_hbm.at[pl.ds(off, 2 * SEQ)], idxbuf.at[pl.ds(2 * SEQ, 2 * SEQ)])


def _load_block_async(src_hbm, trg_hbm, idxbuf, blk_row, isem):
    off = pl.multiple_of(blk_row * SEQ, 8)
    pltpu.async_copy(src_hbm.at[pl.ds(off, 2 * SEQ)],
                     idxbuf.at[pl.ds(0, 2 * SEQ)], isem)
    pltpu.async_copy(trg_hbm.at[pl.ds(off, 2 * SEQ)],
                     idxbuf.at[pl.ds(2 * SEQ, 2 * SEQ)], isem)


def _wait_idx(src_hbm, idxbuf, isem):
    pltpu.make_async_copy(src_hbm.at[pl.ds(0, BI)], idxbuf, isem).wait()


def _drain_all(emb_hbm, rowsbuf, sem):
    pltpu.make_async_copy(emb_hbm.at[pl.ds(0, BI)], rowsbuf, sem).wait()


def _gather_args(emb_hbm, idxbuf, rowsbuf):
    args = []
    for j in range(BI // CH):
        args.append((emb_hbm.at[idxbuf.at[pl.ds(j * CH, CH)]],
                     rowsbuf.at[pl.ds(j * CH, CH)]))
    rem = BI % CH
    if rem:
        args.append((emb_hbm.at[idxbuf.at[pl.ds(BI - rem, rem)]],
                     rowsbuf.at[pl.ds(BI - rem, rem)]))
    return args


def _fire(emb_hbm, idxbuf, rowsbuf, sem):
    for src, dst in _gather_args(emb_hbm, idxbuf, rowsbuf):
        pltpu.async_copy(src, dst, sem)


def _drain(emb_hbm, idxbuf, rowsbuf, sem):
    for src, dst in _gather_args(emb_hbm, idxbuf, rowsbuf):
        pltpu.make_async_copy(src, dst, sem).wait()


def _sum_span(rowsbuf, lo):
    zero = jnp.zeros((LANE,), jnp.float32)

    def body(i, acc):
        a0, a1, a2, a3 = acc
        for u in range(UNROLL):
            r = lo + i * UNROLL + u
            a0 = a0 + rowsbuf[r, pl.ds(0, LANE)]
            a1 = a1 + rowsbuf[r, pl.ds(16, LANE)]
            a2 = a2 + rowsbuf[r, pl.ds(32, LANE)]
            a3 = a3 + rowsbuf[r, pl.ds(48, LANE)]
        return (a0, a1, a2, a3)

    return lax.fori_loop(0, SEQ // UNROLL, body, (zero, zero, zero, zero))


def _count_span(idxbuf, lo):
    one = jnp.ones((LANE,), jnp.float32)
    zero = jnp.zeros((LANE,), jnp.float32)
    c = zero
    for i in range(SEQ // LANE):
        c = c + jnp.where(idxbuf[pl.ds(lo + i * LANE, LANE)] != PAD, one, zero)
    tail = idxbuf[pl.ds(lo + SEQ - LANE, LANE)]
    lane = lax.iota(jnp.int32, LANE)
    c = c + jnp.where((tail != PAD) & (lane >= LANE - SEQ % LANE), one, zero)
    return c


def _consume_row(idxbuf, rowsbuf, wregs, stage, half, rr):
    xs = _sum_span(rowsbuf, half * SEQ)
    ys = _sum_span(rowsbuf, 2 * SEQ + half * SEQ)
    u = jnp.zeros((LANE,), jnp.float32)
    v = jnp.zeros((LANE,), jnp.float32)
    for c in range(4):
        u = u + xs[c] * wregs[c]
        v = v + ys[c] * wregs[4 + c]
    stage[rr, pl.ds(0, LANE)] = u
    stage[rr, pl.ds(LANE, LANE)] = v
    stage[rr, pl.ds(2 * LANE, LANE)] = _count_span(idxbuf, half * SEQ)
    stage[rr, pl.ds(3 * LANE, LANE)] = _count_span(idxbuf, 2 * SEQ + half * SEQ)


def _reduce_group(stage, bsplat, outv, goff):
    lane = lax.iota(jnp.int32, LANE)
    acc = [jnp.zeros((LANE,), jnp.float32) for _ in range(4)]
    for c in range(LANE):
        for q in range(4):
            acc[q] = acc[q] + plsc.load_gather(
                stage, [lane, jnp.full((LANE,), q * LANE + c, jnp.int32)])
    z = acc[0] / acc[2] + acc[1] / acc[3] + bsplat
    plsc.store_scatter(outv, [goff + lane], z)


def _sc_body(src_hbm, trg_hbm, emb_hbm, wb_hbm, out_hbm,
             i0, i1, i2, i3, rows0, rows1, wv, stage, outv,
             semA, semB, isemE, isemO):
    cid = lax.axis_index("c")
    sid = lax.axis_index("s")
    wid = sid * NCORES + cid
    base = wid * BPW

    pltpu.sync_copy(wb_hbm, wv)
    wregs = [wv[pl.ds(c * LANE, LANE)] for c in range(8)]
    bsplat = plsc.load_gather(wv, [jnp.full((LANE,), 2 * DIM, jnp.int32)])

    idxs = (i0, i1, i2, i3)
    rows = (rows0, rows1)
    sems = (semA, semB)
    isems = (isemE, isemO)

    _load_block(src_hbm, trg_hbm, i0, base)
    _fire(emb_hbm, i0, rows0, semA)
    _load_block_async(src_hbm, trg_hbm, i1, base + 2, isemO)
    _load_block_async(src_hbm, trg_hbm, i2, base + 4, isemE)

    def section(c, s):
        p = (s + 1) % 2

        @pl.when(c + 1 < NBLK)
        def _():
            _wait_idx(src_hbm, idxs[(s + 1) % 4], isems[p])
            _fire(emb_hbm, idxs[(s + 1) % 4], rows[p], sems[p])

        @pl.when(c + 3 < NBLK)
        def _():
            _load_block_async(src_hbm, trg_hbm, idxs[(s + 3) % 4],
                              base + 2 * c + 6, isems[p])

        _drain_all(emb_hbm, rows[s % 2], sems[s % 2])
        rr = (2 * c) & (LANE - 1)
        _consume_row(idxs[s], rows[s % 2], wregs, stage, 0, rr)
        _consume_row(idxs[s], rows[s % 2], wregs, stage, 1, rr + 1)

    def ring_body(j, carry):
        for s in range(4):
            section(4 * j + s, s)

        @pl.when((j & 1) == 1)
        def _():
            _reduce_group(stage, bsplat, outv, (j // 2) * LANE)

        return carry

    lax.fori_loop(0, NBLK // 4, ring_body, 0)

    for i in range(BPW // LANE):
        v = outv[pl.ds(i * LANE, LANE)]
        outv[pl.ds(i * LANE, LANE)] = 1.0 / (1.0 + jnp.exp(-v))

    pltpu.sync_copy(outv, out_hbm.at[pl.ds(base, BPW)])


@jax.jit
def kernel(src_tokens, trg_tokens, emb, W, b):
    wb = jnp.concatenate(
        [W.reshape(-1), b.reshape(-1),
         jnp.zeros((15,), jnp.float32)]).astype(jnp.float32)
    mesh = plsc.VectorSubcoreMesh(
        core_axis_name="c", subcore_axis_name="s",
        num_cores=NCORES, num_subcores=NSUB)
    out = pl.kernel(
        _sc_body,
        out_type=jax.ShapeDtypeStruct((BATCH,), jnp.float32),
        mesh=mesh,
        compiler_params=pltpu.CompilerParams(
            needs_layout_passes=False, use_tc_tiling_on_sc=False),
        scratch_types=[
            pltpu.VMEM((BI,), jnp.int32),
            pltpu.VMEM((BI,), jnp.int32),
            pltpu.VMEM((BI,), jnp.int32),
            pltpu.VMEM((BI,), jnp.int32),
            pltpu.VMEM((BI, DIM), jnp.float32),
            pltpu.VMEM((BI, DIM), jnp.float32),
            pltpu.VMEM((2 * DIM + LANE,), jnp.float32),
            pltpu.VMEM((LANE, 4 * LANE), jnp.float32),
            pltpu.VMEM((BPW,), jnp.float32),
            pltpu.SemaphoreType.DMA,
            pltpu.SemaphoreType.DMA,
            pltpu.SemaphoreType.DMA,
            pltpu.SemaphoreType.DMA,
        ],
    )(src_tokens.astype(jnp.int32).reshape(-1),
      trg_tokens.astype(jnp.int32).reshape(-1), emb, wb)
    return out.reshape(BATCH, 1)


# speedup vs baseline: 24.5511x; 1.0764x over previous
import jax
import jax.numpy as jnp
import numpy as np
from jax import lax
from jax.experimental import pallas as pl
from jax.experimental.pallas import tpu as pltpu
from jax.experimental.pallas import tpu_sc as plsc

PAD = 1
BATCH = 4096
SEQ = 200
DIM = 64
NCORES = 2
NSUB = 16
NW = NCORES * NSUB
BPW = BATCH // NW
NBLK = BPW // 2
BI = 4 * SEQ
CH = 128
UNROLL = 8
LANE = 16


def _load_block(src_hbm, trg_hbm, idxbuf, blk_row):
    off = pl.multiple_of(blk_row * SEQ, 8)
    pltpu.sync_copy(src_hbm.at[pl.ds(off, 2 * SEQ)], idxbuf.at[pl.ds(0, 2 * SEQ)])
    pltpu.sync_copy(trg_hbm.at[pl.ds(off, 2 * SEQ)], idxbuf.at[pl.ds(2 * SEQ, 2 * SEQ)])


def _load_block_async(src_hbm, trg_hbm, idxbuf, blk_row, isem):
    off = pl.multiple_of(blk_row * SEQ, 8)
    pltpu.async_copy(src_hbm.at[pl.ds(off, 2 * SEQ)],
                     idxbuf.at[pl.ds(0, 2 * SEQ)], isem)
    pltpu.async_copy(trg_hbm.at[pl.ds(off, 2 * SEQ)],
                     idxbuf.at[pl.ds(2 * SEQ, 2 * SEQ)], isem)


def _wait_idx(src_hbm, idxbuf, isem):
    pltpu.make_async_copy(src_hbm.at[pl.ds(0, BI)], idxbuf, isem).wait()


def _drain_all(emb_hbm, rowsbuf, sem):
    pltpu.make_async_copy(emb_hbm.at[pl.ds(0, BI)], rowsbuf, sem).wait()


def _gather_args(emb_hbm, idxbuf, rowsbuf):
    args = []
    for j in range(BI // CH):
        args.append((emb_hbm.at[idxbuf.at[pl.ds(j * CH, CH)]],
                     rowsbuf.at[pl.ds(j * CH, CH)]))
    rem = BI % CH
    if rem:
        args.append((emb_hbm.at[idxbuf.at[pl.ds(BI - rem, rem)]],
                     rowsbuf.at[pl.ds(BI - rem, rem)]))
    return args


def _fire(emb_hbm, idxbuf, rowsbuf, sem):
    for src, dst in _gather_args(emb_hbm, idxbuf, rowsbuf):
        pltpu.async_copy(src, dst, sem)


def _drain(emb_hbm, idxbuf, rowsbuf, sem):
    for src, dst in _gather_args(emb_hbm, idxbuf, rowsbuf):
        pltpu.make_async_copy(src, dst, sem).wait()


def _sum_span(rowsbuf, lo):
    zero = jnp.zeros((LANE,), jnp.float32)

    def body(i, acc):
        a0, a1, a2, a3 = acc
        for u in range(UNROLL):
            r = lo + i * UNROLL + u
            h0 = rowsbuf[r, pl.ds(0, 2 * LANE)]
            h1 = rowsbuf[r, pl.ds(2 * LANE, 2 * LANE)]
            e0, o0 = plsc.unpack(h0, format=plsc.PackFormat.INTERLEAVED)
            e1, o1 = plsc.unpack(h1, format=plsc.PackFormat.INTERLEAVED)
            a0 = a0 + e0
            a1 = a1 + o0
            a2 = a2 + e1
            a3 = a3 + o1
        return (a0, a1, a2, a3)

    return lax.fori_loop(0, SEQ // UNROLL, body, (zero, zero, zero, zero))


def _count_span(idxbuf, lo):
    one = jnp.ones((LANE,), jnp.float32)
    zero = jnp.zeros((LANE,), jnp.float32)
    c = zero
    for i in range(SEQ // LANE):
        c = c + jnp.where(idxbuf[pl.ds(lo + i * LANE, LANE)] != PAD, one, zero)
    tail = idxbuf[pl.ds(lo + SEQ - LANE, LANE)]
    lane = lax.iota(jnp.int32, LANE)
    c = c + jnp.where((tail != PAD) & (lane >= LANE - SEQ % LANE), one, zero)
    return c


def _consume_row(idxbuf, rowsbuf, wregs, stage, half, rr):
    xs = _sum_span(rowsbuf, half * SEQ)
    ys = _sum_span(rowsbuf, 2 * SEQ + half * SEQ)
    u = jnp.zeros((LANE,), jnp.float32)
    v = jnp.zeros((LANE,), jnp.float32)
    for c in range(4):
        u = u + xs[c] * wregs[c]
        v = v + ys[c] * wregs[4 + c]
    stage[rr, pl.ds(0, LANE)] = u
    stage[rr, pl.ds(LANE, LANE)] = v
    stage[rr, pl.ds(2 * LANE, LANE)] = _count_span(idxbuf, half * SEQ)
    stage[rr, pl.ds(3 * LANE, LANE)] = _count_span(idxbuf, 2 * SEQ + half * SEQ)


def _reduce_group(stage, bsplat, outv, goff):
    lane = lax.iota(jnp.int32, LANE)
    acc = [jnp.zeros((LANE,), jnp.float32) for _ in range(4)]
    for c in range(LANE):
        for q in range(4):
            acc[q] = acc[q] + plsc.load_gather(
                stage, [lane, jnp.full((LANE,), q * LANE + c, jnp.int32)])
    z = acc[0] / acc[2] + acc[1] / acc[3] + bsplat
    plsc.store_scatter(outv, [goff + lane], z)


def _sc_body(src_hbm, trg_hbm, emb_hbm, wb_hbm, out_hbm,
             i0, i1, i2, i3, rows0, rows1, wv, stage, outv,
             semA, semB, isemE, isemO):
    cid = lax.axis_index("c")
    sid = lax.axis_index("s")
    wid = sid * NCORES + cid
    base = wid * BPW

    pltpu.sync_copy(wb_hbm, wv)
    wregs = [wv[pl.ds(c * LANE, LANE)] for c in range(8)]
    bsplat = plsc.load_gather(wv, [jnp.full((LANE,), 2 * DIM, jnp.int32)])

    idxs = (i0, i1, i2, i3)
    rows = (rows0, rows1)
    sems = (semA, semB)
    isems = (isemE, isemO)

    _load_block(src_hbm, trg_hbm, i0, base)
    _fire(emb_hbm, i0, rows0, semA)
    _load_block_async(src_hbm, trg_hbm, i1, base + 2, isemO)
    _load_block_async(src_hbm, trg_hbm, i2, base + 4, isemE)

    def section(c, s):
        p = (s + 1) % 2

        @pl.when(c + 1 < NBLK)
        def _():
            _wait_idx(src_hbm, idxs[(s + 1) % 4], isems[p])
            _fire(emb_hbm, idxs[(s + 1) % 4], rows[p], sems[p])

        @pl.when(c + 3 < NBLK)
        def _():
            _load_block_async(src_hbm, trg_hbm, idxs[(s + 3) % 4],
                              base + 2 * c + 6, isems[p])

        _drain_all(emb_hbm, rows[s % 2], sems[s % 2])
        rr = (2 * c) & (LANE - 1)
        _consume_row(idxs[s], rows[s % 2], wregs, stage, 0, rr)
        _consume_row(idxs[s], rows[s % 2], wregs, stage, 1, rr + 1)

    def ring_body(j, carry):
        for s in range(4):
            section(4 * j + s, s)

        @pl.when((j & 1) == 1)
        def _():
            _reduce_group(stage, bsplat, outv, (j // 2) * LANE)

        return carry

    lax.fori_loop(0, NBLK // 4, ring_body, 0)

    for i in range(BPW // LANE):
        v = outv[pl.ds(i * LANE, LANE)]
        outv[pl.ds(i * LANE, LANE)] = 1.0 / (1.0 + jnp.exp(-v))

    pltpu.sync_copy(outv, out_hbm.at[pl.ds(base, BPW)])


@jax.jit
def kernel(src_tokens, trg_tokens, emb, W, b):
    perm = np.concatenate([np.arange(0, 32, 2), np.arange(1, 32, 2),
                           np.arange(32, 64, 2), np.arange(33, 64, 2)])
    w = W.reshape(-1)
    wb = jnp.concatenate(
        [w[:DIM][perm], w[DIM:][perm], b.reshape(-1),
         jnp.zeros((15,), jnp.float32)]).astype(jnp.float32)
    mesh = plsc.VectorSubcoreMesh(
        core_axis_name="c", subcore_axis_name="s",
        num_cores=NCORES, num_subcores=NSUB)
    out = pl.kernel(
        _sc_body,
        out_type=jax.ShapeDtypeStruct((BATCH,), jnp.float32),
        mesh=mesh,
        compiler_params=pltpu.CompilerParams(
            needs_layout_passes=False, use_tc_tiling_on_sc=False),
        scratch_types=[
            pltpu.VMEM((BI,), jnp.int32),
            pltpu.VMEM((BI,), jnp.int32),
            pltpu.VMEM((BI,), jnp.int32),
            pltpu.VMEM((BI,), jnp.int32),
            pltpu.VMEM((BI, DIM), jnp.bfloat16),
            pltpu.VMEM((BI, DIM), jnp.bfloat16),
            pltpu.VMEM((2 * DIM + LANE,), jnp.float32),
            pltpu.VMEM((LANE, 4 * LANE), jnp.float32),
            pltpu.VMEM((BPW,), jnp.float32),
            pltpu.SemaphoreType.DMA,
            pltpu.SemaphoreType.DMA,
            pltpu.SemaphoreType.DMA,
            pltpu.SemaphoreType.DMA,
        ],
    )(src_tokens.astype(jnp.int32).reshape(-1),
      trg_tokens.astype(jnp.int32).reshape(-1),
      emb.astype(jnp.bfloat16), wb)
    return out.reshape(BATCH, 1)
